# baseline (speedup 1.0000x reference)
import functools

import jax
import jax.numpy as jnp
from jax import lax
from jax.experimental import pallas as pl
from jax.experimental.pallas import tpu as pltpu
from jax.experimental.pallas import tpu_sc as plsc

_NC = 2
_NS = 16
_NW = _NC * _NS

_mesh = plsc.VectorSubcoreMesh(core_axis_name="c", subcore_axis_name="s")


@functools.lru_cache(maxsize=None)
def _make_gather(h_tot, d):
    assert d == 32
    nh = h_tot

    @functools.partial(
        pl.kernel,
        mesh=_mesh,
        compiler_params=pltpu.CompilerParams(use_tc_tiling_on_sc=False, needs_layout_passes=False),
        out_type=jax.ShapeDtypeStruct((nh, 4, _NW, 8, 128), jnp.float32),
        scratch_types=[
            pltpu.VMEM((nh, 128), jnp.int32),
            pltpu.VMEM((2, 128, 32), jnp.float32),
            pltpu.VMEM((32, 129), jnp.float32),
            pltpu.VMEM((32, 129), jnp.float32),
            pltpu.SemaphoreType.DMA,
            pltpu.SemaphoreType.DMA,
        ],
    )
    def gather(table_hbm, idx_hbm, out_hbm, idx_v, rows_v, cb0, cb1, gsem, osem):
        wid = lax.axis_index("s") * _NC + lax.axis_index("c")
        pltpu.sync_copy(idx_hbm.at[wid], idx_v)

        it = lax.iota(jnp.int32, 16)
        w_lo = it
        w_hi = it + 16

        def fire(h, slot):
            pltpu.async_copy(table_hbm.at[idx_v.at[h]], rows_v.at[slot], gsem)

        fire(0, 0)

        @pl.loop(0, nh, step=2)
        def _(h0):
            for s in range(2):
                h = h0 + s
                pltpu.make_async_copy(
                    table_hbm.at[idx_v.at[h]], rows_v.at[s], gsem
                ).wait()

                cb = (cb0, cb1)[s]
                cbo = (cb0, cb1)[1 - s]

                @pl.when(h >= 1)
                def _():
                    for o in range(4):
                        pltpu.make_async_copy(
                            cbo.at[pl.ds(o * 8, 8), pl.ds(0, 128)],
                            out_hbm.at[0, o, wid],
                            osem,
                        ).wait()

                @pl.when(h + 1 < nh)
                def _():
                    fire(h + 1, 1 - s)

                for bl in range(128):
                    blv = it * 0 + bl
                    v0 = rows_v[s, bl, pl.ds(0, 16)]
                    plsc.store_scatter(cb, [w_lo, blv], v0)
                    v1 = rows_v[s, bl, pl.ds(16, 16)]
                    plsc.store_scatter(cb, [w_hi, blv], v1)

                for o in range(4):
                    pltpu.async_copy(
                        cb.at[pl.ds(o * 8, 8), pl.ds(0, 128)],
                        out_hbm.at[h, o, wid],
                        osem,
                    )

        for o in range(4):
            pltpu.make_async_copy(
                (cb0, cb1)[(nh - 1) % 2].at[pl.ds(o * 8, 8), pl.ds(0, 128)],
                out_hbm.at[0, o, wid],
                osem,
            ).wait()

    return gather


_V = 1000000
_NT = _V // 128 + 1
_NFULL = _NT - 1
_VREM = _V - _NFULL * 128


@functools.lru_cache(maxsize=None)
def _make_detile():
    @functools.partial(
        pl.kernel,
        mesh=_mesh,
        compiler_params=pltpu.CompilerParams(needs_layout_passes=False),
        out_type=jax.ShapeDtypeStruct((_NFULL * 32, 128), jnp.float32),
        scratch_types=[
            pltpu.VMEM((32, 128), jnp.float32),
            pltpu.VMEM((32, 128), jnp.float32),
            pltpu.SemaphoreType.DMA,
            pltpu.SemaphoreType.DMA,
        ],
    )
    def detile(wt_hbm, t4_hbm, f0, f1, isem, osem):
        wid = lax.axis_index("s") * _NC + lax.axis_index("c")

        niter = (_NFULL + _NW - 1) // _NW

        @pl.loop(0, niter)
        def _(i):
            blk = wid + _NW * i

            @pl.when(blk < _NFULL)
            def _():
                pltpu.async_copy(
                    wt_hbm.at[:, pl.ds(blk * 128, 128)],
                    t4_hbm.at[pl.ds(blk * 32, 32)],
                    osem,
                )

            @pl.when(jnp.logical_and(i >= 8, blk - 8 * _NW < _NFULL))
            def _():
                pltpu.make_async_copy(
                    wt_hbm.at[:, pl.ds(0, 128)],
                    t4_hbm.at[pl.ds(0, 32)],
                    osem,
                ).wait()

        for j in range(8):
            @pl.when(wid + _NW * (niter - 8 + j) < _NFULL)
            def _():
                pltpu.make_async_copy(
                    wt_hbm.at[:, pl.ds(0, 128)],
                    t4_hbm.at[pl.ds(0, 32)],
                    osem,
                ).wait()

    return detile


@functools.lru_cache(maxsize=None)
def _make_detrans():
    @functools.partial(
        pl.kernel,
        mesh=_mesh,
        compiler_params=pltpu.CompilerParams(
            use_tc_tiling_on_sc=False, needs_layout_passes=False
        ),
        out_type=jax.ShapeDtypeStruct((_V, 32), jnp.float32),
        scratch_types=[
            pltpu.VMEM((32, 128), jnp.float32),
            pltpu.VMEM((32, 128), jnp.float32),
            pltpu.VMEM((128, 33), jnp.float32),
            pltpu.VMEM((128, 33), jnp.float32),
            pltpu.SemaphoreType.DMA,
            pltpu.SemaphoreType.DMA,
        ],
    )
    def detrans(t4_hbm, rem_hbm, t2_hbm, f0, f1, o0, o1, isem, osem):
        wid = lax.axis_index("s") * _NC + lax.axis_index("c")
        it = lax.iota(jnp.int32, 16)

        niter = (_NFULL + _NW - 1) // _NW

        def fire(i, buf):
            blk = wid + _NW * i

            @pl.when(blk < _NFULL)
            def _():
                pltpu.async_copy(t4_hbm.at[pl.ds(blk * 32, 32)], buf, isem)

        fire(0, f0)
        fire(1, f1)

        @pl.loop(0, niter, step=2)
        def _(i0):
            for s in range(2):
                i = i0 + s
                blk = wid + _NW * i
                buf = (f0, f1)[s]
                ob = (o0, o1)[s]

                @pl.when(
                    jnp.logical_and(i >= 2, wid + _NW * (i - 2) < _NFULL)
                )
                def _():
                    pltpu.make_async_copy(
                        ob.at[pl.ds(0, 128), pl.ds(0, 32)],
                        t2_hbm.at[pl.ds(0, 128)],
                        osem,
                    ).wait()

                @pl.when(blk < _NFULL)
                def _():
                    pltpu.make_async_copy(
                        t4_hbm.at[pl.ds(blk * 32, 32)], buf, isem
                    ).wait()

                    for w in range(32):
                        wv = it * 0 + w
                        for g in range(8):
                            v = buf[w, pl.ds(16 * g, 16)]
                            plsc.store_scatter(ob, [16 * g + it, wv], v)

                    pltpu.async_copy(
                        ob.at[pl.ds(0, 128), pl.ds(0, 32)],
                        t2_hbm.at[pl.ds(blk * 128, 128)],
                        osem,
                    )

                fire(i + 2, buf)

        for s in range(2):
            i = niter - 1 + s

            @pl.when(wid + _NW * i < _NFULL)
            def _():
                pltpu.make_async_copy(
                    (o0, o1)[i % 2].at[pl.ds(0, 128), pl.ds(0, 32)],
                    t2_hbm.at[pl.ds(0, 128)],
                    osem,
                ).wait()

        @pl.when(wid == _NW - 1)
        def _():
            pltpu.sync_copy(rem_hbm, t2_hbm.at[pl.ds(_NFULL * 128, _VREM)])

    return detrans


def kernel(X, weights):
    b, h = X.shape
    d = weights.shape[1]
    idx_t = jnp.transpose(
        jnp.reshape(X.astype(jnp.int32), (_NW, b // _NW, h)), (0, 2, 1)
    )
    t4 = _make_detile()(jnp.transpose(weights))
    table = _make_detrans()(t4, weights[_NFULL * 128 :])
    l5 = _make_gather(h, d)(table, idx_t)
    return jnp.reshape(jnp.transpose(l5, (2, 4, 0, 1, 3)), (b, h, d))


# speedup vs baseline: 6.0289x; 6.0289x over previous
import functools

import jax
import jax.numpy as jnp
from jax import lax
from jax.experimental import pallas as pl
from jax.experimental.pallas import tpu as pltpu
from jax.experimental.pallas import tpu_sc as plsc

_NC = 2
_NS = 16
_NW = _NC * _NS

_mesh = plsc.VectorSubcoreMesh(core_axis_name="c", subcore_axis_name="s")


@functools.lru_cache(maxsize=None)
def _make_gather(h_tot, d):
    assert d == 32
    nh = h_tot

    @functools.partial(
        pl.kernel,
        mesh=_mesh,
        compiler_params=pltpu.CompilerParams(use_tc_tiling_on_sc=False, needs_layout_passes=False),
        out_type=jax.ShapeDtypeStruct((nh, 4, _NW, 8, 128), jnp.float32),
        scratch_types=[
            pltpu.VMEM((nh, 128), jnp.int32),
            pltpu.VMEM((2, 128, 32), jnp.float32),
            pltpu.VMEM((32, 129), jnp.float32),
            pltpu.VMEM((32, 129), jnp.float32),
            pltpu.SemaphoreType.DMA,
            pltpu.SemaphoreType.DMA,
        ],
    )
    def gather(table_hbm, idx_hbm, out_hbm, idx_v, rows_v, cb0, cb1, gsem, osem):
        wid = lax.axis_index("s") * _NC + lax.axis_index("c")
        pltpu.sync_copy(idx_hbm.at[wid], idx_v)

        it = lax.iota(jnp.int32, 16)
        w_lo = it
        w_hi = it + 16

        def fire(h, slot):
            pltpu.async_copy(table_hbm.at[idx_v.at[h]], rows_v.at[slot], gsem)

        fire(0, 0)

        @pl.loop(0, nh, step=2)
        def _(h0):
            for s in range(2):
                h = h0 + s
                pltpu.make_async_copy(
                    table_hbm.at[idx_v.at[h]], rows_v.at[s], gsem
                ).wait()

                cb = (cb0, cb1)[s]
                cbo = (cb0, cb1)[1 - s]

                @pl.when(h >= 1)
                def _():
                    for o in range(4):
                        pltpu.make_async_copy(
                            cbo.at[pl.ds(o * 8, 8), pl.ds(0, 128)],
                            out_hbm.at[0, o, wid],
                            osem,
                        ).wait()

                @pl.when(h + 1 < nh)
                def _():
                    fire(h + 1, 1 - s)

                for bl in range(128):
                    blv = it * 0 + bl
                    v0 = rows_v[s, bl, pl.ds(0, 16)]
                    plsc.store_scatter(cb, [w_lo, blv], v0)
                    v1 = rows_v[s, bl, pl.ds(16, 16)]
                    plsc.store_scatter(cb, [w_hi, blv], v1)

                for o in range(4):
                    pltpu.async_copy(
                        cb.at[pl.ds(o * 8, 8), pl.ds(0, 128)],
                        out_hbm.at[h, o, wid],
                        osem,
                    )

        for o in range(4):
            pltpu.make_async_copy(
                (cb0, cb1)[(nh - 1) % 2].at[pl.ds(o * 8, 8), pl.ds(0, 128)],
                out_hbm.at[0, o, wid],
                osem,
            ).wait()

    return gather


def kernel(X, weights):
    b, h = X.shape
    d = weights.shape[1]
    idx_t = jnp.transpose(
        jnp.reshape(X.astype(jnp.int32), (_NW, b // _NW, h)), (0, 2, 1)
    )
    l5 = _make_gather(h, d)(weights, idx_t)
    return jnp.reshape(jnp.transpose(l5, (2, 4, 0, 1, 3)), (b, h, d))
